# baseline (speedup 1.0000x reference)
import jax
import jax.numpy as jnp
from jax import lax
from jax.experimental import pallas as pl
from jax.experimental.pallas import tpu as pltpu
from jax.experimental.pallas import tpu_sc as plsc
import functools

_NODE_NF = 1000000
_HIDDEN = 32
_BATCH = 16384
_FIELDS = 26

_NC = 2
_NS = 16
_NW = _NC * _NS
_RPW = _BATCH // _NW
_NROW = 32
_CHUNK = _NROW * _FIELDS
_NCHUNK = _RPW // _NROW
_NVEC = _CHUNK // 16

_mesh = plsc.VectorSubcoreMesh(core_axis_name="c", subcore_axis_name="s")


@functools.partial(
    pl.kernel,
    mesh=_mesh,
    out_type=jax.ShapeDtypeStruct((_FIELDS, 4, 128, 8, 128), jnp.float32),
    scratch_types=[
        pltpu.VMEM((2 * _NROW, _NROW), jnp.int32),
        pltpu.VMEM((2 * _CHUNK,), jnp.int32),
        pltpu.VMEM((_CHUNK,), jnp.int32),
        pltpu.VMEM((_CHUNK,), jnp.int32),
        pltpu.VMEM((2 * _CHUNK, _HIDDEN), jnp.float32),
        pltpu.VMEM((2 * _NROW, 4, 8, _NROW), jnp.float32),
        pltpu.SemaphoreType.DMA,
        pltpu.SemaphoreType.DMA,
        pltpu.SemaphoreType.DMA,
    ],
    compiler_params=pltpu.CompilerParams(use_tc_tiling_on_sc=False,
                                         needs_layout_passes=False),
)
def _gather_kernel(zt_hbm, table_hbm, nidx_hbm, fidx_hbm, out_hbm, z2_v,
                   idx_v, nid_v, fid_v, gbuf_v, wbuf_v, sem_i, sem_g, sem_w):
    wid = lax.axis_index("s") * _NC + lax.axis_index("c")
    row0 = wid * _RPW

    pltpu.sync_copy(nidx_hbm, nid_v)
    pltpu.sync_copy(fidx_hbm, fid_v)
    nstep = lax.iota(jnp.int32, 16) * _FIELDS

    def issue_z2(g):
        b = (g & 1) * _NROW
        pltpu.async_copy(
            zt_hbm.at[:, pl.ds(row0 + g * _NROW, _NROW)],
            z2_v.at[pl.ds(b, _FIELDS)], sem_i)

    def flatten(g):
        b = (g & 1) * _NROW
        pltpu.make_async_copy(
            zt_hbm.at[:, pl.ds(0, _NROW)], z2_v.at[pl.ds(0, _FIELDS)],
            sem_i).wait()

        def body(t, carry):
            nn = nid_v[pl.ds(t * 16, 16)]
            ff = fid_v[pl.ds(t * 16, 16)] + b
            idx_v[pl.ds((g & 1) * _CHUNK + t * 16, 16)] = plsc.load_gather(
                z2_v, [ff, nn])
            return carry

        lax.fori_loop(0, _NVEC, body, 0)

    def fire_gather(g):
        b = (g & 1) * _CHUNK
        pltpu.async_copy(
            table_hbm.at[idx_v.at[pl.ds(b, _CHUNK)]],
            gbuf_v.at[pl.ds(b, _CHUNK)], sem_g)

    def wait_gather():
        pltpu.make_async_copy(
            table_hbm.at[pl.ds(0, _CHUNK)], gbuf_v.at[pl.ds(0, _CHUNK)],
            sem_g).wait()

    def repack(g):
        goff = (g & 1) * _CHUNK
        woff = (g & 1) * _NROW

        def body(f, carry):
            for hb in range(4):
                for hs in range(8):
                    cols = jnp.full((16,), hb * 8 + hs, jnp.int32)
                    for half in range(2):
                        rows = nstep + (goff + f + half * 16 * _FIELDS)
                        wbuf_v[woff + f, hb, hs, pl.ds(half * 16, 16)] = (
                            plsc.load_gather(gbuf_v, [rows, cols]))
            return carry

        lax.fori_loop(0, _FIELDS, body, 0)

    def fire_wb(g):
        b = (g & 1) * _NROW
        nb = wid * 4 + lax.shift_right_logical(g, 2)
        nl0 = lax.bitwise_and(g, 3) * _NROW
        pltpu.async_copy(
            wbuf_v.at[pl.ds(b, _FIELDS)],
            out_hbm.at[:, :, nb, :, pl.ds(nl0, _NROW)], sem_w)

    def drain_wb():
        pltpu.make_async_copy(
            out_hbm.at[:, :, 0, :, pl.ds(0, _NROW)],
            wbuf_v.at[pl.ds(0, _FIELDS)], sem_w).wait()

    issue_z2(0)
    issue_z2(1)
    flatten(0)
    fire_gather(0)

    def chunk_body(g, carry):
        @pl.when(g + 1 < _NCHUNK)
        def _():
            flatten(g + 1)
            fire_gather(g + 1)

        wait_gather()

        @pl.when(g >= 2)
        def _():
            drain_wb()

        repack(g)
        fire_wb(g)

        @pl.when(g + 2 < _NCHUNK)
        def _():
            issue_z2(g + 2)

        return carry

    lax.fori_loop(0, _NCHUNK, chunk_body, 0)
    drain_wb()
    drain_wb()


def kernel(z, table):
    jj = jnp.arange(_CHUNK, dtype=jnp.int32)
    out5 = _gather_kernel(z.T, table, jj // _FIELDS, jj % _FIELDS)
    emb = out5.transpose(2, 4, 0, 1, 3).reshape(_BATCH, _FIELDS, _HIDDEN)
    return (emb, 0)


# speedup vs baseline: 1.0164x; 1.0164x over previous
import jax
import jax.numpy as jnp
from jax import lax
from jax.experimental import pallas as pl
from jax.experimental.pallas import tpu as pltpu
from jax.experimental.pallas import tpu_sc as plsc
import functools

_NODE_NF = 1000000
_HIDDEN = 32
_BATCH = 16384
_FIELDS = 26

_NC = 2
_NS = 16
_NW = _NC * _NS
_RPW = _BATCH // _NW
_NROW = 32
_CHUNK = _NROW * _FIELDS
_NCHUNK = _RPW // _NROW

_mesh = plsc.VectorSubcoreMesh(core_axis_name="c", subcore_axis_name="s")


@functools.partial(
    pl.kernel,
    mesh=_mesh,
    out_type=jax.ShapeDtypeStruct((_FIELDS, 4, 128, 8, 128), jnp.float32),
    scratch_types=[
        pltpu.VMEM((2 * _CHUNK,), jnp.int32),
        pltpu.VMEM((2 * _CHUNK, _HIDDEN), jnp.float32),
        pltpu.VMEM((2 * _NROW, 4, 8, _NROW), jnp.float32),
        pltpu.SemaphoreType.DMA,
        pltpu.SemaphoreType.DMA,
        pltpu.SemaphoreType.DMA,
    ],
    compiler_params=pltpu.CompilerParams(use_tc_tiling_on_sc=False,
                                         needs_layout_passes=False),
)
def _gather_kernel(zt_hbm, table_hbm, out_hbm, idx_v, gbuf_v, wbuf_v,
                   sem_i, sem_g, sem_w):
    wid = lax.axis_index("s") * _NC + lax.axis_index("c")
    row0 = wid * _RPW
    riota = lax.iota(jnp.int32, 16)

    def issue_idx(g):
        b = (g & 1) * _CHUNK
        n0 = row0 + g * _NROW
        for f in range(_FIELDS):
            pltpu.async_copy(
                zt_hbm.at[f, pl.ds(n0, _NROW)],
                idx_v.at[pl.ds(b + f * _NROW, _NROW)], sem_i)

    def drain_idx():
        pltpu.make_async_copy(
            table_hbm.at[pl.ds(0, _FIELDS)], gbuf_v.at[pl.ds(0, _FIELDS)],
            sem_i).wait()

    def fire_gather(g):
        b = (g & 1) * _CHUNK
        pltpu.async_copy(
            table_hbm.at[idx_v.at[pl.ds(b, _CHUNK)]],
            gbuf_v.at[pl.ds(b, _CHUNK)], sem_g)

    def wait_gather():
        pltpu.make_async_copy(
            table_hbm.at[pl.ds(0, _CHUNK)], gbuf_v.at[pl.ds(0, _CHUNK)],
            sem_g).wait()

    def repack(g):
        goff = (g & 1) * _CHUNK
        woff = (g & 1) * _NROW

        def body(f, carry):
            base = goff + f * _NROW
            for half in range(2):
                rows = riota + (base + half * 16)
                for h in range(_HIDDEN):
                    wbuf_v[woff + f, h // 8, h % 8,
                           pl.ds(half * 16, 16)] = plsc.load_gather(
                               gbuf_v, [rows, jnp.full((16,), h, jnp.int32)])
            return carry

        lax.fori_loop(0, _FIELDS, body, 0)

    def fire_wb(g):
        b = (g & 1) * _NROW
        nb = wid * 4 + lax.shift_right_logical(g, 2)
        nl0 = lax.bitwise_and(g, 3) * _NROW
        pltpu.async_copy(
            wbuf_v.at[pl.ds(b, _FIELDS)],
            out_hbm.at[:, :, nb, :, pl.ds(nl0, _NROW)], sem_w)

    def drain_wb():
        pltpu.make_async_copy(
            out_hbm.at[:, :, 0, :, pl.ds(0, _NROW)],
            wbuf_v.at[pl.ds(0, _FIELDS)], sem_w).wait()

    issue_idx(0)
    issue_idx(1)
    drain_idx()
    fire_gather(0)

    def chunk_body(g, carry):
        @pl.when(g + 1 < _NCHUNK)
        def _():
            drain_idx()
            fire_gather(g + 1)

        wait_gather()

        @pl.when(g >= 2)
        def _():
            drain_wb()

        repack(g)
        fire_wb(g)

        @pl.when(g + 2 < _NCHUNK)
        def _():
            issue_idx(g + 2)

        return carry

    lax.fori_loop(0, _NCHUNK, chunk_body, 0)
    drain_wb()
    drain_wb()


def kernel(z, table):
    out5 = _gather_kernel(z.T, table)
    emb = out5.transpose(2, 4, 0, 1, 3).reshape(_BATCH, _FIELDS, _HIDDEN)
    return (emb, 0)


# speedup vs baseline: 1.3151x; 1.2939x over previous
import jax
import jax.numpy as jnp
from jax import lax
from jax.experimental import pallas as pl
from jax.experimental.pallas import tpu as pltpu
from jax.experimental.pallas import tpu_sc as plsc
import functools

_NODE_NF = 1000000
_HIDDEN = 32
_BATCH = 16384
_FIELDS = 26

_NC = 2
_NS = 16
_NW = _NC * _NS
_RPW = _BATCH // _NW
_NROW = 32
_CHUNK = _NROW * _FIELDS
_NCHUNK = _RPW // _NROW

_mesh = plsc.VectorSubcoreMesh(core_axis_name="c", subcore_axis_name="s")


@functools.partial(
    pl.kernel,
    mesh=_mesh,
    out_type=jax.ShapeDtypeStruct((_FIELDS, 4, 128, 8, 128), jnp.float32),
    scratch_types=[
        pltpu.VMEM((2 * _CHUNK,), jnp.int32),
        pltpu.VMEM((2 * _CHUNK, _HIDDEN), jnp.float32),
        pltpu.VMEM((2 * _NROW, 4, 8, _NROW), jnp.float32),
        pltpu.SemaphoreType.DMA,
        pltpu.SemaphoreType.DMA,
        pltpu.SemaphoreType.DMA,
    ],
    compiler_params=pltpu.CompilerParams(use_tc_tiling_on_sc=False,
                                         needs_layout_passes=False),
)
def _gather_kernel(zt_hbm, table_hbm, out_hbm, idx_v, gbuf_v, wbuf_v,
                   sem_i, sem_g, sem_w):
    wid = lax.axis_index("s") * _NC + lax.axis_index("c")
    row0 = wid * _RPW
    riota = lax.iota(jnp.int32, 16)

    def issue_idx(g):
        b = (g & 1) * _CHUNK
        n0 = row0 + g * _NROW
        for f in range(_FIELDS):
            pltpu.async_copy(
                zt_hbm.at[f, pl.ds(n0, _NROW)],
                idx_v.at[pl.ds(b + f * _NROW, _NROW)], sem_i)

    def drain_idx():
        pltpu.make_async_copy(
            table_hbm.at[pl.ds(0, _FIELDS)], gbuf_v.at[pl.ds(0, _FIELDS)],
            sem_i).wait()

    def fire_gather(g):
        b = (g & 1) * _CHUNK
        pltpu.async_copy(
            table_hbm.at[idx_v.at[pl.ds(b, _CHUNK)]],
            gbuf_v.at[pl.ds(b, _CHUNK)], sem_g)

    def wait_gather():
        pltpu.make_async_copy(
            table_hbm.at[pl.ds(0, _CHUNK)], gbuf_v.at[pl.ds(0, _CHUNK)],
            sem_g).wait()

    def repack(g):
        goff = (g & 1) * _CHUNK
        woff = (g & 1) * _NROW

        def body(f, carry):
            base = goff + f * _NROW
            for half in range(2):
                rows = riota + (base + half * 16)
                nl = riota + half * 16
                for k in range(_HIDDEN):
                    hh = (riota + k) & (_HIDDEN - 1)
                    vals = plsc.load_gather(gbuf_v, [rows, hh])
                    plsc.store_scatter(
                        wbuf_v,
                        [jnp.full((16,), woff + f, jnp.int32),
                         lax.shift_right_logical(hh, 3),
                         lax.bitwise_and(hh, 7), nl],
                        vals)
            return carry

        lax.fori_loop(0, _FIELDS, body, 0)

    def fire_wb(g):
        b = (g & 1) * _NROW
        nb = wid * 4 + lax.shift_right_logical(g, 2)
        nl0 = lax.bitwise_and(g, 3) * _NROW
        pltpu.async_copy(
            wbuf_v.at[pl.ds(b, _FIELDS)],
            out_hbm.at[:, :, nb, :, pl.ds(nl0, _NROW)], sem_w)

    def drain_wb():
        pltpu.make_async_copy(
            out_hbm.at[:, :, 0, :, pl.ds(0, _NROW)],
            wbuf_v.at[pl.ds(0, _FIELDS)], sem_w).wait()

    issue_idx(0)
    issue_idx(1)
    drain_idx()
    fire_gather(0)

    def chunk_body(g, carry):
        @pl.when(g + 1 < _NCHUNK)
        def _():
            drain_idx()
            fire_gather(g + 1)

        wait_gather()

        @pl.when(g >= 2)
        def _():
            drain_wb()

        repack(g)
        fire_wb(g)

        @pl.when(g + 2 < _NCHUNK)
        def _():
            issue_idx(g + 2)

        return carry

    lax.fori_loop(0, _NCHUNK, chunk_body, 0)
    drain_wb()
    drain_wb()


def kernel(z, table):
    out5 = _gather_kernel(z.T, table)
    emb = out5.transpose(2, 4, 0, 1, 3).reshape(_BATCH, _FIELDS, _HIDDEN)
    return (emb, 0)
